# baseline (speedup 1.0000x reference)
import functools

import jax
import jax.numpy as jnp
from jax import lax
from jax.experimental import pallas as pl
from jax.experimental.pallas import tpu as pltpu
from jax.experimental.pallas import tpu_sc as plsc

_N = 10000
_E = 320000
_D = 128
_L = 4


def _mm_body(x_ref, w_ref, b_ref, o_ref):
    o_ref[...] = (
        jnp.dot(x_ref[...], w_ref[...], preferred_element_type=jnp.float32)
        + b_ref[...]
    )


def _matmul(x, w, b, block_rows):
    rows, k = x.shape
    dout = w.shape[1]
    return pl.pallas_call(
        _mm_body,
        grid=(rows // block_rows,),
        in_specs=[
            pl.BlockSpec((block_rows, k), lambda i: (i, 0)),
            pl.BlockSpec((k, dout), lambda i: (0, 0)),
            pl.BlockSpec((1, dout), lambda i: (0, 0)),
        ],
        out_specs=pl.BlockSpec((block_rows, dout), lambda i: (i, 0)),
        out_shape=jax.ShapeDtypeStruct((rows, dout), jnp.float32),
    )(x, w, b.reshape(1, dout))


def _mm_split_body(x_ref, w_ref, b_ref, o_ref):
    o_ref[0] = (
        jnp.dot(x_ref[...], w_ref[0], preferred_element_type=jnp.float32)
        + b_ref[0]
    )


def _matmul_pair(x, wsp, bsp, block_rows):
    rows, k = x.shape
    dout = wsp.shape[2]
    return pl.pallas_call(
        _mm_split_body,
        grid=(rows // block_rows, 2),
        in_specs=[
            pl.BlockSpec((block_rows, k), lambda i, c: (i, 0)),
            pl.BlockSpec((1, k, dout), lambda i, c: (c, 0, 0)),
            pl.BlockSpec((1, 1, dout), lambda i, c: (c, 0, 0)),
        ],
        out_specs=pl.BlockSpec((1, block_rows, dout), lambda i, c: (c, i, 0)),
        out_shape=jax.ShapeDtypeStruct((2, rows, dout), jnp.float32),
    )(x, wsp, bsp)


def _matmul_split(x, w, b, block_rows):
    k = x.shape[1]
    wsp = w.reshape(k, 2, 64).transpose(1, 0, 2)
    bsp = b.reshape(2, 1, 64)
    return _matmul_pair(x, wsp, bsp, block_rows)


_CB = 80
_EPT = _E // 16
_NCH = _EPT // _CB


def _sc_edge(bd2, eh2, ce, sdi):
    mesh = plsc.VectorSubcoreMesh(core_axis_name="c", subcore_axis_name="s")
    out_type = [
        jax.ShapeDtypeStruct((2, _E, 64), jnp.float32),
        jax.ShapeDtypeStruct((2, _N, _D), jnp.float32),
        jax.ShapeDtypeStruct((2, 16, _D), jnp.float32),
    ]
    scratch_types = (
        [pltpu.VMEM((4 * _CB,), jnp.int32)]
        + [pltpu.VMEM((_CB,), jnp.int32) for _ in range(2)]
        + [pltpu.VMEM((_CB,), jnp.int32) for _ in range(2)]
        + [pltpu.VMEM((_CB,), jnp.int32) for _ in range(2)]
        + [pltpu.VMEM((_CB, _D), jnp.float32) for _ in range(2)]
        + [pltpu.VMEM((_CB, 64), jnp.float32) for _ in range(2)]
        + [pltpu.VMEM((_CB, 64), jnp.float32) for _ in range(2)]
        + [pltpu.SemaphoreType.DMA for _ in range(20)]
        + [
            pltpu.VMEM((128,), jnp.float32),
            pltpu.VMEM_SHARED((_N, _D), jnp.float32),
        ]
    )

    @functools.partial(
        pl.kernel,
        out_type=out_type,
        mesh=mesh,
        scratch_types=scratch_types,
        compiler_params=pltpu.CompilerParams(use_tc_tiling_on_sc=False),
    )
    def k(bd_hbm, eh_hbm, ce_hbm, sdi_hbm, enew_hbm, nd_hbm, st_hbm, *scr):
        sdp = scr[0]
        bdi = scr[1:3]
        edi = scr[3:5]
        dsc = scr[5:7]
        bd = scr[7:9]
        eh = scr[9:11]
        cv = scr[11:13]
        sems = scr[13:33]
        statv, ndacc = scr[33:35]

        c = lax.axis_index("c")
        s = lax.axis_index("s")
        cN = c * _N
        zero16 = jnp.zeros((16,), jnp.float32)

        zv = bd[0]

        @pl.loop(0, _CB)
        def _(r):
            for k8 in range(8):
                zv[r, pl.ds(k8 * 16, 16)] = zero16

        for k8 in range(8):
            statv[pl.ds(k8 * 16, 16)] = zero16

        @pl.when(s < 15)
        def _():
            @pl.loop(0, 640 // _CB)
            def _(q):
                pltpu.sync_copy(zv, ndacc.at[pl.ds(s * 640 + q * _CB, _CB)])

        @pl.when(s == 15)
        def _():
            @pl.loop(0, 400 // _CB)
            def _(q):
                pltpu.sync_copy(zv, ndacc.at[pl.ds(9600 + q * _CB, _CB)])

        plsc.subcore_barrier()

        idx_row0 = s * (_NCH // 2)

        def build(q):
            off = 2 * _CB * q
            for j in range(_CB // 16):
                sl = pl.ds(j * 16, 16)
                s0 = sdp[pl.ds(off + j * 16, 16)]
                d0 = sdp[pl.ds(off + _CB + j * 16, 16)]
                bdi[q][sl] = s0 + cN
                edi[q][sl] = d0 + cN
                dsc[q][sl] = d0

        def fire_in(q, g):
            base = s * _EPT + g * _CB
            sq = sems[10 * q : 10 * q + 8]
            hs = []
            for j in range(5):
                hs.append(
                    pltpu.async_copy(
                        bd_hbm.at[bdi[q].at[pl.ds(j * 16, 16)]],
                        bd[q].at[pl.ds(j * 16, 16)],
                        sq[j],
                    )
                )
            for j in range(2):
                hs.append(
                    pltpu.async_copy(
                        eh_hbm.at[edi[q].at[pl.ds(j * 40, 40)]],
                        eh[q].at[pl.ds(j * 40, 40)],
                        sq[5 + j],
                    )
                )
            hs.append(
                pltpu.async_copy(
                    ce_hbm.at[c].at[pl.ds(base, _CB)], cv[q], sq[7]
                )
            )
            return hs

        def compute(q):
            cvq, bdq, ehq = cv[q], bd[q], eh[q]

            @pl.loop(0, _CB)
            def _(r):
                for k4 in range(4):
                    sl = pl.ds(k4 * 16, 16)
                    sh = pl.ds(64 + k4 * 16, 16)
                    en = cvq[r, sl] + bdq[r, sh] + ehq[r, sl]
                    cvq[r, sl] = en
                    sg = 1.0 / (1.0 + jnp.exp(-en))
                    nc = sg * bdq[r, sl]
                    bdq[r, sl] = nc
                    bdq[r, sh] = sg
                    statv[sl] = statv[sl] + en
                    statv[sh] = statv[sh] + en * en

        def fire_out(q, g):
            base = s * _EPT + g * _CB
            sq = sems[10 * q + 8 : 10 * q + 10]
            return [
                pltpu.async_copy(
                    cv[q], enew_hbm.at[c].at[pl.ds(base, _CB)], sq[0]
                ),
                pltpu.async_copy(bd[q], ndacc.at[dsc[q]], sq[1], add=True),
            ]

        def wait_in(q):
            sq = sems[10 * q : 10 * q + 8]
            for j in range(5):
                pltpu.make_async_copy(
                    bd_hbm.at[bdi[q].at[pl.ds(j * 16, 16)]],
                    bd[q].at[pl.ds(j * 16, 16)],
                    sq[j],
                ).wait()
            for j in range(2):
                pltpu.make_async_copy(
                    eh_hbm.at[edi[q].at[pl.ds(j * 40, 40)]],
                    eh[q].at[pl.ds(j * 40, 40)],
                    sq[5 + j],
                ).wait()
            pltpu.make_async_copy(
                ce_hbm.at[0].at[pl.ds(0, _CB)], cv[q], sq[7]
            ).wait()

        pltpu.sync_copy(sdi_hbm.at[idx_row0], sdp)
        build(0)
        fire_in(0, 0)
        build(1)
        fire_in(1, 1)

        @pl.loop(0, _NCH // 2)
        def _(i):
            wait_in(0)
            compute(0)
            h_out_a = fire_out(0, 2 * i)
            wait_in(1)
            compute(1)
            h_out_b = fire_out(1, 2 * i + 1)
            for h in h_out_a + h_out_b:
                h.wait()

            @pl.when(i + 1 < _NCH // 2)
            def _():
                pltpu.sync_copy(sdi_hbm.at[idx_row0 + i + 1], sdp)
                build(0)
                fire_in(0, 2 * i + 2)
                build(1)
                fire_in(1, 2 * i + 3)

        plsc.subcore_barrier()

        @pl.when(s < 15)
        def _():
            row0 = s * 640
            pltpu.sync_copy(
                ndacc.at[pl.ds(row0, 640)], nd_hbm.at[c].at[pl.ds(row0, 640)]
            )

        @pl.when(s == 15)
        def _():
            pltpu.sync_copy(
                ndacc.at[pl.ds(9600, 400)], nd_hbm.at[c].at[pl.ds(9600, 400)]
            )

        pltpu.sync_copy(statv, st_hbm.at[c].at[s])

    return k(bd2, eh2, ce, sdi)


def _h_update(h_in, ah, nd, gamma, beta):
    def body(h_ref, ah_ref, nd_ref, g_ref, b_ref, o_ref):
        nda = nd_ref[...]
        num2 = jnp.concatenate([nda[0, :, :64], nda[1, :, :64]], axis=1)
        den2 = jnp.concatenate([nda[0, :, 64:], nda[1, :, 64:]], axis=1)
        h_new = ah_ref[...] + num2 / (den2 + 1e-6)
        mu = jnp.mean(h_new, axis=0, keepdims=True)
        var = jnp.mean((h_new - mu) ** 2, axis=0, keepdims=True)
        bn = (h_new - mu) * lax.rsqrt(var + 1e-5) * g_ref[...] + b_ref[...]
        o_ref[...] = h_ref[...] + jnp.maximum(bn, 0.0)

    return pl.pallas_call(
        body,
        in_specs=[
            pl.BlockSpec((_N, _D), lambda: (0, 0)),
            pl.BlockSpec((_N, _D), lambda: (0, 0)),
            pl.BlockSpec((2, _N, _D), lambda: (0, 0, 0)),
            pl.BlockSpec((1, _D), lambda: (0, 0)),
            pl.BlockSpec((1, _D), lambda: (0, 0)),
        ],
        out_specs=pl.BlockSpec((_N, _D), lambda: (0, 0)),
        out_shape=jax.ShapeDtypeStruct((_N, _D), jnp.float32),
    )(h_in, ah, nd, gamma.reshape(1, _D), beta.reshape(1, _D))


_BEU = 2000


def _e_update(e_in, e_new, scale, shift):
    def body(e_ref, lo_ref, hi_ref, sc_ref, sh_ref, o_ref):
        en = jnp.concatenate([lo_ref[0], hi_ref[0]], axis=1)
        o_ref[...] = e_ref[...] + jnp.maximum(
            en * sc_ref[...] + sh_ref[...], 0.0
        )

    return pl.pallas_call(
        body,
        grid=(_E // _BEU,),
        in_specs=[
            pl.BlockSpec((_BEU, _D), lambda i: (i, 0)),
            pl.BlockSpec((1, _BEU, 64), lambda i: (0, i, 0)),
            pl.BlockSpec((1, _BEU, 64), lambda i: (1, i, 0)),
            pl.BlockSpec((1, _D), lambda i: (0, 0)),
            pl.BlockSpec((1, _D), lambda i: (0, 0)),
        ],
        out_specs=pl.BlockSpec((_BEU, _D), lambda i: (i, 0)),
        out_shape=jax.ShapeDtypeStruct((_E, _D), jnp.float32),
    )(e_in, e_new, e_new, scale, shift)


def _readout(h4, w1, b1, w2, b2, w3p, b3p):
    def body(h_ref, w1_ref, b1_ref, w2_ref, b2_ref, w3_ref, b3_ref, o_ref):
        y = jnp.mean(h_ref[...], axis=0, keepdims=True)
        y = jnp.broadcast_to(y, (8, _D))
        y = jnp.maximum(
            jnp.dot(y, w1_ref[...], preferred_element_type=jnp.float32)
            + b1_ref[...],
            0.0,
        )
        y = jnp.maximum(
            jnp.dot(y, w2_ref[...], preferred_element_type=jnp.float32)
            + b2_ref[...],
            0.0,
        )
        o_ref[...] = (
            jnp.dot(y, w3_ref[...], preferred_element_type=jnp.float32)
            + b3_ref[...]
        )

    return pl.pallas_call(
        body,
        in_specs=[
            pl.BlockSpec((_N, _D), lambda: (0, 0)),
            pl.BlockSpec((_D, _D), lambda: (0, 0)),
            pl.BlockSpec((1, _D), lambda: (0, 0)),
            pl.BlockSpec((_D, _D), lambda: (0, 0)),
            pl.BlockSpec((1, _D), lambda: (0, 0)),
            pl.BlockSpec((_D, _D), lambda: (0, 0)),
            pl.BlockSpec((1, _D), lambda: (0, 0)),
        ],
        out_specs=pl.BlockSpec((8, _D), lambda: (0, 0)),
        out_shape=jax.ShapeDtypeStruct((8, _D), jnp.float32),
    )(
        h4,
        w1,
        b1.reshape(1, _D),
        w2,
        b2.reshape(1, _D),
        w3p,
        b3p.reshape(1, _D),
    )


def kernel(
    h,
    e,
    edge_index,
    W_emb_h,
    b_emb_h,
    W_emb_e,
    b_emb_e,
    W_A,
    b_A,
    W_B,
    b_B,
    W_C,
    b_C,
    W_D,
    b_D,
    W_E,
    b_E,
    gamma_h,
    beta_h,
    gamma_e,
    beta_e,
    W1,
    b1,
    W2,
    b2,
    W3,
    b3,
):
    src = edge_index[0].astype(jnp.int32)
    dst = edge_index[1].astype(jnp.int32)
    srcr = src.reshape(16, _NCH // 2, 2, _CB)
    dstr = dst.reshape(16, _NCH // 2, 2, _CB)
    sdi = jnp.stack(
        [srcr[:, :, 0], dstr[:, :, 0], srcr[:, :, 1], dstr[:, :, 1]], axis=2
    ).reshape(16 * (_NCH // 2), 4 * _CB)

    h = _matmul(h, W_emb_h, b_emb_h, 2000)
    e = _matmul(e, W_emb_e, b_emb_e, 2000)

    for l in range(_L):
        ah = _matmul(h, W_A[l], b_A[l], 2000)
        wbd = jnp.stack(
            [
                jnp.concatenate([W_B[l][:, :64], W_D[l][:, :64]], axis=1),
                jnp.concatenate([W_B[l][:, 64:], W_D[l][:, 64:]], axis=1),
            ]
        )
        bbd = jnp.stack(
            [
                jnp.concatenate([b_B[l][:64], b_D[l][:64]]),
                jnp.concatenate([b_B[l][64:], b_D[l][64:]]),
            ]
        ).reshape(2, 1, _D)
        bd = _matmul_pair(h, wbd, bbd, 2000).reshape(2 * _N, _D)
        ehs = _matmul_split(h, W_E[l], b_E[l], 2000).reshape(2 * _N, 64)
        ce = _matmul_split(e, W_C[l], b_C[l], 2000)
        e_new, nd, st = _sc_edge(bd, ehs, ce, sdi)
        h = _h_update(h, ah, nd, gamma_h[l], beta_h[l])
        if l < _L - 1:
            cnt = float(_E)
            ssum = jnp.concatenate(
                [st[0, :, :64].sum(axis=0), st[1, :, :64].sum(axis=0)]
            )
            ssq = jnp.concatenate(
                [st[0, :, 64:].sum(axis=0), st[1, :, 64:].sum(axis=0)]
            )
            mu = ssum / cnt
            var = ssq / cnt - mu * mu
            rstd = lax.rsqrt(var + 1e-5)
            scale = (gamma_e[l] * rstd).reshape(1, _D)
            shift = (beta_e[l] - mu * rstd * gamma_e[l]).reshape(1, _D)
            e = _e_update(e, e_new, scale, shift)

    w3p = jnp.zeros((_D, _D), jnp.float32).at[:, :10].set(W3)
    b3p = jnp.zeros((_D,), jnp.float32).at[:10].set(b3)
    y = _readout(h, W1, b1, W2, b2, w3p, b3p)
    return y[0:1, 0:10]


# speedup vs baseline: 1.0764x; 1.0764x over previous
import functools

import jax
import jax.numpy as jnp
from jax import lax
from jax.experimental import pallas as pl
from jax.experimental.pallas import tpu as pltpu
from jax.experimental.pallas import tpu_sc as plsc

_N = 10000
_E = 320000
_D = 128
_L = 4


def _mm_body(x_ref, w_ref, b_ref, o_ref):
    o_ref[...] = (
        jnp.dot(x_ref[...], w_ref[...], preferred_element_type=jnp.float32)
        + b_ref[...]
    )


def _matmul(x, w, b, block_rows):
    rows, k = x.shape
    dout = w.shape[1]
    return pl.pallas_call(
        _mm_body,
        grid=(rows // block_rows,),
        in_specs=[
            pl.BlockSpec((block_rows, k), lambda i: (i, 0)),
            pl.BlockSpec((k, dout), lambda i: (0, 0)),
            pl.BlockSpec((1, dout), lambda i: (0, 0)),
        ],
        out_specs=pl.BlockSpec((block_rows, dout), lambda i: (i, 0)),
        out_shape=jax.ShapeDtypeStruct((rows, dout), jnp.float32),
    )(x, w, b.reshape(1, dout))


def _mm_split_body(x_ref, w_ref, b_ref, o_ref):
    o_ref[0] = (
        jnp.dot(x_ref[...], w_ref[0], preferred_element_type=jnp.float32)
        + b_ref[0]
    )


def _matmul_pair(x, wsp, bsp, block_rows):
    rows, k = x.shape
    dout = wsp.shape[2]
    return pl.pallas_call(
        _mm_split_body,
        grid=(rows // block_rows, 2),
        in_specs=[
            pl.BlockSpec((block_rows, k), lambda i, c: (i, 0)),
            pl.BlockSpec((1, k, dout), lambda i, c: (c, 0, 0)),
            pl.BlockSpec((1, 1, dout), lambda i, c: (c, 0, 0)),
        ],
        out_specs=pl.BlockSpec((1, block_rows, dout), lambda i, c: (c, i, 0)),
        out_shape=jax.ShapeDtypeStruct((2, rows, dout), jnp.float32),
    )(x, wsp, bsp)


def _matmul_split(x, w, b, block_rows):
    k = x.shape[1]
    wsp = w.reshape(k, 2, 64).transpose(1, 0, 2)
    bsp = b.reshape(2, 1, 64)
    return _matmul_pair(x, wsp, bsp, block_rows)


_CB = 80
_EPT = _E // 16
_NCH = _EPT // _CB


def _sc_edge(bd2, eh2, ce, sdi):
    mesh = plsc.VectorSubcoreMesh(core_axis_name="c", subcore_axis_name="s")
    out_type = [
        jax.ShapeDtypeStruct((2, _E, 64), jnp.float32),
        jax.ShapeDtypeStruct((2, _N, _D), jnp.float32),
    ]
    scratch_types = (
        [pltpu.VMEM((4 * _CB,), jnp.int32)]
        + [pltpu.VMEM((_CB,), jnp.int32) for _ in range(2)]
        + [pltpu.VMEM((_CB,), jnp.int32) for _ in range(2)]
        + [pltpu.VMEM((_CB,), jnp.int32) for _ in range(2)]
        + [pltpu.VMEM((_CB, _D), jnp.float32) for _ in range(2)]
        + [pltpu.VMEM((_CB, 64), jnp.float32) for _ in range(2)]
        + [pltpu.VMEM((_CB, 64), jnp.float32) for _ in range(2)]
        + [pltpu.SemaphoreType.DMA for _ in range(20)]
        + [pltpu.VMEM_SHARED((_N, _D), jnp.float32)]
    )

    @functools.partial(
        pl.kernel,
        out_type=out_type,
        mesh=mesh,
        scratch_types=scratch_types,
        compiler_params=pltpu.CompilerParams(use_tc_tiling_on_sc=False),
    )
    def k(bd_hbm, eh_hbm, ce_hbm, sdi_hbm, enew_hbm, nd_hbm, *scr):
        sdp = scr[0]
        bdi = scr[1:3]
        edi = scr[3:5]
        dsc = scr[5:7]
        bd = scr[7:9]
        eh = scr[9:11]
        cv = scr[11:13]
        sems = scr[13:33]
        ndacc = scr[33]

        c = lax.axis_index("c")
        s = lax.axis_index("s")
        cN = c * _N
        zero16 = jnp.zeros((16,), jnp.float32)

        zv = bd[0]

        @pl.loop(0, _CB)
        def _(r):
            for k8 in range(8):
                zv[r, pl.ds(k8 * 16, 16)] = zero16

        @pl.when(s < 15)
        def _():
            @pl.loop(0, 640 // _CB)
            def _(q):
                pltpu.sync_copy(zv, ndacc.at[pl.ds(s * 640 + q * _CB, _CB)])

        @pl.when(s == 15)
        def _():
            @pl.loop(0, 400 // _CB)
            def _(q):
                pltpu.sync_copy(zv, ndacc.at[pl.ds(9600 + q * _CB, _CB)])

        plsc.subcore_barrier()

        idx_row0 = s * (_NCH // 2)

        def build(q):
            off = 2 * _CB * q
            for j in range(_CB // 16):
                sl = pl.ds(j * 16, 16)
                s0 = sdp[pl.ds(off + j * 16, 16)]
                d0 = sdp[pl.ds(off + _CB + j * 16, 16)]
                bdi[q][sl] = s0 + cN
                edi[q][sl] = d0 + cN
                dsc[q][sl] = d0

        def fire_in(q, g):
            base = s * _EPT + g * _CB
            sq = sems[10 * q : 10 * q + 8]
            hs = []
            for j in range(5):
                hs.append(
                    pltpu.async_copy(
                        bd_hbm.at[bdi[q].at[pl.ds(j * 16, 16)]],
                        bd[q].at[pl.ds(j * 16, 16)],
                        sq[j],
                    )
                )
            for j in range(2):
                hs.append(
                    pltpu.async_copy(
                        eh_hbm.at[edi[q].at[pl.ds(j * 40, 40)]],
                        eh[q].at[pl.ds(j * 40, 40)],
                        sq[5 + j],
                    )
                )
            hs.append(
                pltpu.async_copy(
                    ce_hbm.at[c].at[pl.ds(base, _CB)], cv[q], sq[7]
                )
            )
            return hs

        def compute(q):
            cvq, bdq, ehq = cv[q], bd[q], eh[q]

            @pl.loop(0, _CB // 2)
            def _(r2):
                for u in range(2):
                    r = r2 * 2 + u
                    for k4 in range(4):
                        sl = pl.ds(k4 * 16, 16)
                        sh = pl.ds(64 + k4 * 16, 16)
                        en = cvq[r, sl] + bdq[r, sh] + ehq[r, sl]
                        cvq[r, sl] = en
                        sg = 1.0 / (1.0 + jnp.exp(-en))
                        nc = sg * bdq[r, sl]
                        bdq[r, sl] = nc
                        bdq[r, sh] = sg

        def fire_out(q, g):
            base = s * _EPT + g * _CB
            sq = sems[10 * q + 8 : 10 * q + 10]
            return [
                pltpu.async_copy(
                    cv[q], enew_hbm.at[c].at[pl.ds(base, _CB)], sq[0]
                ),
                pltpu.async_copy(bd[q], ndacc.at[dsc[q]], sq[1], add=True),
            ]

        def wait_in(q):
            sq = sems[10 * q : 10 * q + 8]
            for j in range(5):
                pltpu.make_async_copy(
                    bd_hbm.at[bdi[q].at[pl.ds(j * 16, 16)]],
                    bd[q].at[pl.ds(j * 16, 16)],
                    sq[j],
                ).wait()
            for j in range(2):
                pltpu.make_async_copy(
                    eh_hbm.at[edi[q].at[pl.ds(j * 40, 40)]],
                    eh[q].at[pl.ds(j * 40, 40)],
                    sq[5 + j],
                ).wait()
            pltpu.make_async_copy(
                ce_hbm.at[0].at[pl.ds(0, _CB)], cv[q], sq[7]
            ).wait()

        pltpu.sync_copy(sdi_hbm.at[idx_row0], sdp)
        build(0)
        fire_in(0, 0)
        build(1)
        fire_in(1, 1)

        @pl.loop(0, _NCH // 2)
        def _(i):
            wait_in(0)
            compute(0)
            h_out_a = fire_out(0, 2 * i)
            wait_in(1)
            compute(1)
            h_out_b = fire_out(1, 2 * i + 1)
            for h in h_out_a + h_out_b:
                h.wait()

            @pl.when(i + 1 < _NCH // 2)
            def _():
                pltpu.sync_copy(sdi_hbm.at[idx_row0 + i + 1], sdp)
                build(0)
                fire_in(0, 2 * i + 2)
                build(1)
                fire_in(1, 2 * i + 3)

        plsc.subcore_barrier()

        @pl.when(s < 15)
        def _():
            row0 = s * 640
            pltpu.sync_copy(
                ndacc.at[pl.ds(row0, 640)], nd_hbm.at[c].at[pl.ds(row0, 640)]
            )

        @pl.when(s == 15)
        def _():
            pltpu.sync_copy(
                ndacc.at[pl.ds(9600, 400)], nd_hbm.at[c].at[pl.ds(9600, 400)]
            )

    return k(bd2, eh2, ce, sdi)


def _h_update(h_in, ah, nd, gamma, beta):
    def body(h_ref, ah_ref, nd_ref, g_ref, b_ref, o_ref):
        nda = nd_ref[...]
        num2 = jnp.concatenate([nda[0, :, :64], nda[1, :, :64]], axis=1)
        den2 = jnp.concatenate([nda[0, :, 64:], nda[1, :, 64:]], axis=1)
        h_new = ah_ref[...] + num2 / (den2 + 1e-6)
        mu = jnp.mean(h_new, axis=0, keepdims=True)
        var = jnp.mean((h_new - mu) ** 2, axis=0, keepdims=True)
        bn = (h_new - mu) * lax.rsqrt(var + 1e-5) * g_ref[...] + b_ref[...]
        o_ref[...] = h_ref[...] + jnp.maximum(bn, 0.0)

    return pl.pallas_call(
        body,
        in_specs=[
            pl.BlockSpec((_N, _D), lambda: (0, 0)),
            pl.BlockSpec((_N, _D), lambda: (0, 0)),
            pl.BlockSpec((2, _N, _D), lambda: (0, 0, 0)),
            pl.BlockSpec((1, _D), lambda: (0, 0)),
            pl.BlockSpec((1, _D), lambda: (0, 0)),
        ],
        out_specs=pl.BlockSpec((_N, _D), lambda: (0, 0)),
        out_shape=jax.ShapeDtypeStruct((_N, _D), jnp.float32),
    )(h_in, ah, nd, gamma.reshape(1, _D), beta.reshape(1, _D))


_BEU = 2000


def _e_stats(e_new):
    nblk = _E // _BEU

    def body(x_ref, o_ref):
        i = pl.program_id(1)
        x = x_ref[0]
        blk = jnp.concatenate(
            [
                jnp.sum(x, axis=0, keepdims=True),
                jnp.sum(x * x, axis=0, keepdims=True),
            ],
            axis=0,
        )

        @pl.when(i == 0)
        def _():
            o_ref[0] = blk

        @pl.when(i > 0)
        def _():
            o_ref[0] = o_ref[0] + blk

    return pl.pallas_call(
        body,
        grid=(2, nblk),
        in_specs=[pl.BlockSpec((1, _BEU, 64), lambda c, i: (c, i, 0))],
        out_specs=pl.BlockSpec((1, 2, 64), lambda c, i: (c, 0, 0)),
        out_shape=jax.ShapeDtypeStruct((2, 2, 64), jnp.float32),
    )(e_new)


def _e_update(e_in, e_new, scale, shift):
    def body(e_ref, lo_ref, hi_ref, sc_ref, sh_ref, o_ref):
        en = jnp.concatenate([lo_ref[0], hi_ref[0]], axis=1)
        o_ref[...] = e_ref[...] + jnp.maximum(
            en * sc_ref[...] + sh_ref[...], 0.0
        )

    return pl.pallas_call(
        body,
        grid=(_E // _BEU,),
        in_specs=[
            pl.BlockSpec((_BEU, _D), lambda i: (i, 0)),
            pl.BlockSpec((1, _BEU, 64), lambda i: (0, i, 0)),
            pl.BlockSpec((1, _BEU, 64), lambda i: (1, i, 0)),
            pl.BlockSpec((1, _D), lambda i: (0, 0)),
            pl.BlockSpec((1, _D), lambda i: (0, 0)),
        ],
        out_specs=pl.BlockSpec((_BEU, _D), lambda i: (i, 0)),
        out_shape=jax.ShapeDtypeStruct((_E, _D), jnp.float32),
    )(e_in, e_new, e_new, scale, shift)


def _readout(h4, w1, b1, w2, b2, w3p, b3p):
    def body(h_ref, w1_ref, b1_ref, w2_ref, b2_ref, w3_ref, b3_ref, o_ref):
        y = jnp.mean(h_ref[...], axis=0, keepdims=True)
        y = jnp.broadcast_to(y, (8, _D))
        y = jnp.maximum(
            jnp.dot(y, w1_ref[...], preferred_element_type=jnp.float32)
            + b1_ref[...],
            0.0,
        )
        y = jnp.maximum(
            jnp.dot(y, w2_ref[...], preferred_element_type=jnp.float32)
            + b2_ref[...],
            0.0,
        )
        o_ref[...] = (
            jnp.dot(y, w3_ref[...], preferred_element_type=jnp.float32)
            + b3_ref[...]
        )

    return pl.pallas_call(
        body,
        in_specs=[
            pl.BlockSpec((_N, _D), lambda: (0, 0)),
            pl.BlockSpec((_D, _D), lambda: (0, 0)),
            pl.BlockSpec((1, _D), lambda: (0, 0)),
            pl.BlockSpec((_D, _D), lambda: (0, 0)),
            pl.BlockSpec((1, _D), lambda: (0, 0)),
            pl.BlockSpec((_D, _D), lambda: (0, 0)),
            pl.BlockSpec((1, _D), lambda: (0, 0)),
        ],
        out_specs=pl.BlockSpec((8, _D), lambda: (0, 0)),
        out_shape=jax.ShapeDtypeStruct((8, _D), jnp.float32),
    )(
        h4,
        w1,
        b1.reshape(1, _D),
        w2,
        b2.reshape(1, _D),
        w3p,
        b3p.reshape(1, _D),
    )


def kernel(
    h,
    e,
    edge_index,
    W_emb_h,
    b_emb_h,
    W_emb_e,
    b_emb_e,
    W_A,
    b_A,
    W_B,
    b_B,
    W_C,
    b_C,
    W_D,
    b_D,
    W_E,
    b_E,
    gamma_h,
    beta_h,
    gamma_e,
    beta_e,
    W1,
    b1,
    W2,
    b2,
    W3,
    b3,
):
    src = edge_index[0].astype(jnp.int32)
    dst = edge_index[1].astype(jnp.int32)
    srcr = src.reshape(16, _NCH // 2, 2, _CB)
    dstr = dst.reshape(16, _NCH // 2, 2, _CB)
    sdi = jnp.stack(
        [srcr[:, :, 0], dstr[:, :, 0], srcr[:, :, 1], dstr[:, :, 1]], axis=2
    ).reshape(16 * (_NCH // 2), 4 * _CB)

    h = _matmul(h, W_emb_h, b_emb_h, 2000)
    e = _matmul(e, W_emb_e, b_emb_e, 2000)

    for l in range(_L):
        ah = _matmul(h, W_A[l], b_A[l], 2000)
        wbd = jnp.stack(
            [
                jnp.concatenate([W_B[l][:, :64], W_D[l][:, :64]], axis=1),
                jnp.concatenate([W_B[l][:, 64:], W_D[l][:, 64:]], axis=1),
            ]
        )
        bbd = jnp.stack(
            [
                jnp.concatenate([b_B[l][:64], b_D[l][:64]]),
                jnp.concatenate([b_B[l][64:], b_D[l][64:]]),
            ]
        ).reshape(2, 1, _D)
        bd = _matmul_pair(h, wbd, bbd, 2000).reshape(2 * _N, _D)
        ehs = _matmul_split(h, W_E[l], b_E[l], 2000).reshape(2 * _N, 64)
        ce = _matmul_split(e, W_C[l], b_C[l], 2000)
        e_new, nd = _sc_edge(bd, ehs, ce, sdi)
        h = _h_update(h, ah, nd, gamma_h[l], beta_h[l])
        if l < _L - 1:
            cnt = float(_E)
            st = _e_stats(e_new)
            ssum = jnp.concatenate([st[0, 0], st[1, 0]])
            ssq = jnp.concatenate([st[0, 1], st[1, 1]])
            mu = ssum / cnt
            var = ssq / cnt - mu * mu
            rstd = lax.rsqrt(var + 1e-5)
            scale = (gamma_e[l] * rstd).reshape(1, _D)
            shift = (beta_e[l] - mu * rstd * gamma_e[l]).reshape(1, _D)
            e = _e_update(e, e_new, scale, shift)

    w3p = jnp.zeros((_D, _D), jnp.float32).at[:, :10].set(W3)
    b3p = jnp.zeros((_D,), jnp.float32).at[:10].set(b3)
    y = _readout(h, W1, b1, W2, b2, w3p, b3p)
    return y[0:1, 0:10]
